# baseline (speedup 1.0000x reference)
import functools

import jax
import jax.numpy as jnp
from jax import lax
from jax.experimental import pallas as pl
from jax.experimental.pallas import tpu as pltpu
from jax.experimental.pallas import tpu_sc as plsc

V = 1000000
D = 64
C = 200
B = 4096
L = 16
NC, NS = 2, 16
NW = NC * NS
VBLK = 128
NFULL = V // VBLK
VTAIL = V - NFULL * VBLK
BBLK = 128

_mesh = plsc.VectorSubcoreMesh(core_axis_name="c", subcore_axis_name="s")


@functools.partial(
    pl.kernel,
    out_type=jax.ShapeDtypeStruct((V, 128), jnp.float32),
    mesh=_mesh,
    scratch_types=[
        pltpu.VMEM((D, VBLK), jnp.float32),
        pltpu.VMEM((VBLK, 128), jnp.float32),
        pltpu.SemaphoreType.DMA,
    ],
    compiler_params=pltpu.CompilerParams(use_tc_tiling_on_sc=True, needs_layout_passes=False),
)
def _format_table(tablet_hbm, tail_hbm, out_hbm, a_v, t_v, sem):
    wid = lax.axis_index("s") * NC + lax.axis_index("c")
    nblk = 244 + (wid < NFULL - 244 * NW).astype(jnp.int32)

    iota = lax.broadcasted_iota(jnp.int32, (L,), 0)

    @pl.when(wid == 0)
    def _():
        pltpu.sync_copy(tail_hbm, out_hbm.at[pl.ds(NFULL * VBLK, VTAIL)])

    def blk_body(k, _):
        i0 = (wid + k * NW) * VBLK
        pltpu.async_copy(
            tablet_hbm.at[:, pl.ds(i0, VBLK)], a_v, sem).wait()

        def row_body(i, _):
            col = jnp.full((L,), 0, jnp.int32) + i
            for dk in range(D // L):
                dvec = iota + (dk * L)
                t_v[i, pl.ds(dk * L, L)] = plsc.load_gather(a_v, [dvec, col])
            return 0

        lax.fori_loop(0, VBLK, row_body, 0)
        pltpu.sync_copy(t_v, out_hbm.at[pl.ds(i0, VBLK)])
        return 0

    lax.fori_loop(0, nblk, blk_body, 0)


@functools.partial(
    pl.kernel,
    out_type=jax.ShapeDtypeStruct((C, 8, NW, 8, 128), jnp.float32),
    mesh=_mesh,
    scratch_types=[
        pltpu.VMEM((C, BBLK), jnp.int32),
        pltpu.VMEM((2, BBLK, 128), jnp.float32),
        pltpu.VMEM((2, 8, 8, 128), jnp.float32),
        pltpu.SemaphoreType.DMA,
        pltpu.SemaphoreType.DMA,
        pltpu.SemaphoreType.DMA,
        pltpu.SemaphoreType.DMA,
    ],
    compiler_params=pltpu.CompilerParams(use_tc_tiling_on_sc=True, needs_layout_passes=False),
)
def _embed(idxt_hbm, tableg_hbm, out_hbm, idx_v, g_v, o_v, g0, g1, o0, o1):
    wid = lax.axis_index("s") * NC + lax.axis_index("c")
    pltpu.sync_copy(idxt_hbm.at[:, pl.ds(wid * BBLK, BBLK)], idx_v)

    gsems = (g0, g1)
    osems = (o0, o1)
    iota = lax.broadcasted_iota(jnp.int32, (L,), 0)

    def gather(c, b):
        pltpu.async_copy(tableg_hbm.at[idx_v.at[c]], g_v.at[b], gsems[b])

    def wait_gather(b):
        pltpu.make_async_copy(
            tableg_hbm.at[pl.ds(0, BBLK)], g_v.at[b], gsems[b]).wait()

    def wait_out(b):
        pltpu.make_async_copy(out_hbm.at[0, :, 0], o_v.at[b], osems[b]).wait()

    gather(0, 0)
    gather(1, 1)

    def group_body(g, _):
        for b in range(2):
            c = 2 * g + b
            wait_gather(b)

            @pl.when(g >= 1)
            def _():
                wait_out(b)

            p = lax.convert_element_type(c + 1, jnp.float32) * (1.0 / C)
            c1 = 1.0 - p
            c2 = 1.0 - 2.0 * p

            def bb_body(b0, _):
                bvec = iota + b0 * L
                for dh in range(8):
                    for dl in range(8):
                        d = dh * 8 + dl
                        pe = c1 - ((d + 1) * (1.0 / D)) * c2
                        dvec = jnp.full((L,), d, jnp.int32)
                        vec = plsc.load_gather(g_v.at[b], [bvec, dvec])
                        o_v[b, dh, dl, pl.ds(b0 * L, L)] = vec + pe
                return 0

            lax.fori_loop(0, BBLK // L, bb_body, 0)

            @pl.when(g < C // 2 - 1)
            def _():
                gather(c + 2, b)

            pltpu.async_copy(o_v.at[b], out_hbm.at[c, :, wid], osems[b])
        return 0

    lax.fori_loop(0, C // 2, group_body, 0)

    wait_out(0)
    wait_out(1)


def _tail_pad(table):
    tail = table[NFULL * VBLK:, :]
    return jnp.pad(tail, ((0, 0), (0, 128 - D)))


def kernel(inputs, table):
    idxt = inputs.T.astype(jnp.int32)
    tablet = table.T
    tailp = _tail_pad(table)
    tg = _format_table(tablet, tailp)
    out5 = _embed(idxt, tg)
    return out5.transpose(2, 4, 0, 1, 3).reshape(B, C, D)
